# baseline (speedup 1.0000x reference)
import functools

import jax
import jax.numpy as jnp
from jax import lax
from jax.experimental import pallas as pl
from jax.experimental.pallas import tpu as pltpu
from jax.experimental.pallas import tpu_sc as plsc

VOCAB = 100000
EMB_DIM = 1024
BATCH = 4096
BAG = 20


ROWS_PER_STEP = 2000


def _project_body(t1_ref, t2_ref, w1aT_ref, w1bT_ref, w2aT_ref, w2bT_ref,
                  b1a_ref, b1b_ref, b2a_ref, b2b_ref, s1_ref, s2_ref):
    v1 = jnp.dot(w1bT_ref[...], w1aT_ref[...], preferred_element_type=jnp.float32)
    v2 = jnp.dot(w2bT_ref[...], w2aT_ref[...], preferred_element_type=jnp.float32)
    c = (jnp.sum(b1a_ref[...] * w1bT_ref[...])
         + jnp.sum(b2a_ref[...] * w2bT_ref[...]))
    off = (c + b1b_ref[0, 0] + b2b_ref[0, 0]) * (1.0 / (2.0 * BAG))
    s1_ref[...] = jnp.sum(t1_ref[...] * v1, axis=1, keepdims=True) + off
    s2_ref[...] = jnp.sum(t2_ref[...] * (v1 + v2), axis=1, keepdims=True) + off


def _project_tables(table_1, table_2, w1a, w1b, w2a, w2b, b1a, b1b, b2a, b2b):
    grid = (VOCAB // ROWS_PER_STEP,)
    full = lambda shape: pl.BlockSpec(shape, lambda i: (0, 0))
    s1, s2 = pl.pallas_call(
        _project_body,
        grid=grid,
        in_specs=[
            pl.BlockSpec((ROWS_PER_STEP, EMB_DIM), lambda i: (i, 0)),
            pl.BlockSpec((ROWS_PER_STEP, EMB_DIM), lambda i: (i, 0)),
            full((512, EMB_DIM)),
            full((1, 512)),
            full((512, EMB_DIM)),
            full((1, 512)),
            full((1, 512)),
            full((1, 1)),
            full((1, 512)),
            full((1, 1)),
        ],
        out_specs=[
            pl.BlockSpec((ROWS_PER_STEP, 1), lambda i: (i, 0)),
            pl.BlockSpec((ROWS_PER_STEP, 1), lambda i: (i, 0)),
        ],
        out_shape=[
            jax.ShapeDtypeStruct((VOCAB, 1), jnp.float32),
            jax.ShapeDtypeStruct((VOCAB, 1), jnp.float32),
        ],
    )(table_1, table_2, w1a.T, w1b.T, w2a.T, w2b.T,
      b1a.reshape(1, 512), b1b.reshape(1, 1),
      b2a.reshape(1, 512), b2b.reshape(1, 1))
    return s1.reshape(VOCAB), s2.reshape(VOCAB)


NUM_CORES = 2
NUM_SUBCORES = 16
NUM_WORKERS = NUM_CORES * NUM_SUBCORES
ROWS_PER_WORKER = BATCH // NUM_WORKERS


def _sc_body(s1_hbm, s2_hbm, xt_hbm, yt_hbm, out_hbm, idx_v, vals_v, out_v, sem):
    wid = lax.axis_index("s") * NUM_CORES + lax.axis_index("c")
    base = wid * ROWS_PER_WORKER

    pltpu.sync_copy(xt_hbm.at[wid], idx_v)
    descs = []
    for j in range(BAG):
        descs.append(pltpu.async_copy(s1_hbm.at[idx_v.at[j]], vals_v.at[j], sem))
    for d in descs:
        d.wait()
    for k in range(ROWS_PER_WORKER // 16):
        sl = pl.ds(k * 16, 16)
        acc = vals_v[0, sl]
        for j in range(1, BAG):
            acc = acc + vals_v[j, sl]
        out_v[sl] = acc

    pltpu.sync_copy(yt_hbm.at[wid], idx_v)
    descs = []
    for j in range(BAG):
        descs.append(pltpu.async_copy(s2_hbm.at[idx_v.at[j]], vals_v.at[j], sem))
    for d in descs:
        d.wait()
    for k in range(ROWS_PER_WORKER // 16):
        sl = pl.ds(k * 16, 16)
        acc = out_v[sl]
        for j in range(BAG):
            acc = acc + vals_v[j, sl]
        out_v[sl] = acc

    pltpu.sync_copy(out_v, out_hbm.at[pl.ds(base, ROWS_PER_WORKER)])


@functools.lru_cache(maxsize=None)
def _sc_bag_sum():
    return functools.partial(
        pl.kernel,
        mesh=plsc.VectorSubcoreMesh(core_axis_name="c", subcore_axis_name="s"),
        out_type=jax.ShapeDtypeStruct((BATCH,), jnp.float32),
        scratch_types=[
            pltpu.VMEM((BAG, ROWS_PER_WORKER), jnp.int32),
            pltpu.VMEM((BAG, ROWS_PER_WORKER), jnp.float32),
            pltpu.VMEM((ROWS_PER_WORKER,), jnp.float32),
            pltpu.SemaphoreType.DMA,
        ],
    )(_sc_body)


def kernel(x, y, table_1, table_2, w1a, b1a, w1b, b1b, w2a, b2a, w2b, b2b):
    s1, s2 = _project_tables(table_1, table_2, w1a, w1b, w2a, w2b,
                             b1a, b1b, b2a, b2b)
    xt = x.reshape(NUM_WORKERS, ROWS_PER_WORKER, BAG).transpose(0, 2, 1)
    yt = y.reshape(NUM_WORKERS, ROWS_PER_WORKER, BAG).transpose(0, 2, 1)
    out = _sc_bag_sum()(s1, s2, xt, yt)
    return out.reshape(BATCH, 1)


# speedup vs baseline: 1.0007x; 1.0007x over previous
import functools

import jax
import jax.numpy as jnp
from jax import lax
from jax.experimental import pallas as pl
from jax.experimental.pallas import tpu as pltpu
from jax.experimental.pallas import tpu_sc as plsc

VOCAB = 100000
EMB_DIM = 1024
BATCH = 4096
BAG = 20


ROWS_PER_STEP = 2000


HALF = EMB_DIM // 2


def _project_body(t1L_ref, t1R_ref, t2L_ref, t2R_ref,
                  w1aT_ref, w1bT_ref, w2aT_ref, w2bT_ref,
                  b1a_ref, b1b_ref, b2a_ref, b2b_ref, s1_ref, s2_ref):
    v1 = jnp.dot(w1bT_ref[...], w1aT_ref[...], preferred_element_type=jnp.float32)
    v2 = jnp.dot(w2bT_ref[...], w2aT_ref[...], preferred_element_type=jnp.float32)
    c = (jnp.sum(b1a_ref[...] * w1bT_ref[...])
         + jnp.sum(b2a_ref[...] * w2bT_ref[...]))
    off = (c + b1b_ref[0, 0] + b2b_ref[0, 0]) * (1.0 / (2.0 * BAG))
    v12 = v1 + v2
    s1_ref[...] = (jnp.sum(t1L_ref[...] * v1[:, :HALF], axis=1, keepdims=True)
                   + jnp.sum(t1R_ref[...] * v1[:, HALF:], axis=1, keepdims=True)
                   + off)
    s2_ref[...] = (jnp.sum(t2L_ref[...] * v12[:, :HALF], axis=1, keepdims=True)
                   + jnp.sum(t2R_ref[...] * v12[:, HALF:], axis=1, keepdims=True)
                   + off)


def _project_tables(table_1, table_2, w1a, w1b, w2a, w2b, b1a, b1b, b2a, b2b):
    grid = (VOCAB // ROWS_PER_STEP,)
    full = lambda shape: pl.BlockSpec(shape, lambda i: (0, 0))
    half_l = pl.BlockSpec((ROWS_PER_STEP, HALF), lambda i: (i, 0))
    half_r = pl.BlockSpec((ROWS_PER_STEP, HALF), lambda i: (i, 1))
    s1, s2 = pl.pallas_call(
        _project_body,
        grid=grid,
        in_specs=[
            half_l, half_r, half_l, half_r,
            full((512, EMB_DIM)),
            full((1, 512)),
            full((512, EMB_DIM)),
            full((1, 512)),
            full((1, 512)),
            full((1, 1)),
            full((1, 512)),
            full((1, 1)),
        ],
        out_specs=[
            pl.BlockSpec((ROWS_PER_STEP, 1), lambda i: (i, 0)),
            pl.BlockSpec((ROWS_PER_STEP, 1), lambda i: (i, 0)),
        ],
        out_shape=[
            jax.ShapeDtypeStruct((VOCAB, 1), jnp.float32),
            jax.ShapeDtypeStruct((VOCAB, 1), jnp.float32),
        ],
    )(table_1, table_1, table_2, table_2, w1a.T, w1b.T, w2a.T, w2b.T,
      b1a.reshape(1, 512), b1b.reshape(1, 1),
      b2a.reshape(1, 512), b2b.reshape(1, 1))
    return s1.reshape(VOCAB), s2.reshape(VOCAB)


NUM_CORES = 2
NUM_SUBCORES = 16
NUM_WORKERS = NUM_CORES * NUM_SUBCORES
ROWS_PER_WORKER = BATCH // NUM_WORKERS


def _sc_body(s1_hbm, s2_hbm, xt_hbm, yt_hbm, out_hbm, idx_v, vals_v, out_v, sem):
    wid = lax.axis_index("s") * NUM_CORES + lax.axis_index("c")
    base = wid * ROWS_PER_WORKER

    pltpu.sync_copy(xt_hbm.at[wid], idx_v)
    descs = []
    for j in range(BAG):
        descs.append(pltpu.async_copy(s1_hbm.at[idx_v.at[j]], vals_v.at[j], sem))
    for d in descs:
        d.wait()
    for k in range(ROWS_PER_WORKER // 16):
        sl = pl.ds(k * 16, 16)
        acc = vals_v[0, sl]
        for j in range(1, BAG):
            acc = acc + vals_v[j, sl]
        out_v[sl] = acc

    pltpu.sync_copy(yt_hbm.at[wid], idx_v)
    descs = []
    for j in range(BAG):
        descs.append(pltpu.async_copy(s2_hbm.at[idx_v.at[j]], vals_v.at[j], sem))
    for d in descs:
        d.wait()
    for k in range(ROWS_PER_WORKER // 16):
        sl = pl.ds(k * 16, 16)
        acc = out_v[sl]
        for j in range(BAG):
            acc = acc + vals_v[j, sl]
        out_v[sl] = acc

    pltpu.sync_copy(out_v, out_hbm.at[pl.ds(base, ROWS_PER_WORKER)])


@functools.lru_cache(maxsize=None)
def _sc_bag_sum():
    return functools.partial(
        pl.kernel,
        mesh=plsc.VectorSubcoreMesh(core_axis_name="c", subcore_axis_name="s"),
        out_type=jax.ShapeDtypeStruct((BATCH,), jnp.float32),
        scratch_types=[
            pltpu.VMEM((BAG, ROWS_PER_WORKER), jnp.int32),
            pltpu.VMEM((BAG, ROWS_PER_WORKER), jnp.float32),
            pltpu.VMEM((ROWS_PER_WORKER,), jnp.float32),
            pltpu.SemaphoreType.DMA,
        ],
    )(_sc_body)


def kernel(x, y, table_1, table_2, w1a, b1a, w1b, b1b, w2a, b2a, w2b, b2b):
    s1, s2 = _project_tables(table_1, table_2, w1a, w1b, w2a, w2b,
                             b1a, b1b, b2a, b2b)
    xt = x.reshape(NUM_WORKERS, ROWS_PER_WORKER, BAG).transpose(0, 2, 1)
    yt = y.reshape(NUM_WORKERS, ROWS_PER_WORKER, BAG).transpose(0, 2, 1)
    out = _sc_bag_sum()(s1, s2, xt, yt)
    return out.reshape(BATCH, 1)


# speedup vs baseline: 1.0023x; 1.0016x over previous
import functools

import jax
import jax.numpy as jnp
from jax import lax
from jax.experimental import pallas as pl
from jax.experimental.pallas import tpu as pltpu
from jax.experimental.pallas import tpu_sc as plsc

VOCAB = 100000
EMB_DIM = 1024
BATCH = 4096
BAG = 20


ROWS_PER_STEP = 2000


def _project_body(t1_ref, t2_ref, w1aT_ref, w1bT_ref, w2aT_ref, w2bT_ref,
                  b1a_ref, b1b_ref, b2a_ref, b2b_ref, s1_ref, s2_ref):
    v1 = jnp.dot(w1bT_ref[...], w1aT_ref[...], preferred_element_type=jnp.float32)
    v2 = jnp.dot(w2bT_ref[...], w2aT_ref[...], preferred_element_type=jnp.float32)
    c = (jnp.sum(b1a_ref[...] * w1bT_ref[...])
         + jnp.sum(b2a_ref[...] * w2bT_ref[...]))
    off = (c + b1b_ref[0, 0] + b2b_ref[0, 0]) * (1.0 / (2.0 * BAG))
    s1_ref[...] = jnp.sum(t1_ref[...] * v1, axis=1, keepdims=True) + off
    s2_ref[...] = jnp.sum(t2_ref[...] * (v1 + v2), axis=1, keepdims=True) + off


def _project_tables(table_1, table_2, w1a, w1b, w2a, w2b, b1a, b1b, b2a, b2b):
    grid = (VOCAB // ROWS_PER_STEP,)
    full = lambda shape: pl.BlockSpec(shape, lambda i: (0, 0))
    s1, s2 = pl.pallas_call(
        _project_body,
        grid=grid,
        in_specs=[
            pl.BlockSpec((ROWS_PER_STEP, EMB_DIM), lambda i: (i, 0)),
            pl.BlockSpec((ROWS_PER_STEP, EMB_DIM), lambda i: (i, 0)),
            full((512, EMB_DIM)),
            full((1, 512)),
            full((512, EMB_DIM)),
            full((1, 512)),
            full((1, 512)),
            full((1, 1)),
            full((1, 512)),
            full((1, 1)),
        ],
        out_specs=[
            pl.BlockSpec((ROWS_PER_STEP, 1), lambda i: (i, 0)),
            pl.BlockSpec((ROWS_PER_STEP, 1), lambda i: (i, 0)),
        ],
        out_shape=[
            jax.ShapeDtypeStruct((VOCAB, 1), jnp.float32),
            jax.ShapeDtypeStruct((VOCAB, 1), jnp.float32),
        ],
    )(table_1, table_2, w1a.T, w1b.T, w2a.T, w2b.T,
      b1a.reshape(1, 512), b1b.reshape(1, 1),
      b2a.reshape(1, 512), b2b.reshape(1, 1))
    return s1.reshape(VOCAB), s2.reshape(VOCAB)


NUM_CORES = 2
NUM_SUBCORES = 16
NUM_WORKERS = NUM_CORES * NUM_SUBCORES
ROWS_PER_WORKER = BATCH // NUM_WORKERS


def _sc_body(s1_hbm, s2_hbm, xt_hbm, yt_hbm, out_hbm,
             idx1_v, idx2_v, vals1_v, vals2_v, out_v, sem):
    wid = lax.axis_index("s") * NUM_CORES + lax.axis_index("c")
    base = wid * ROWS_PER_WORKER

    pltpu.sync_copy(xt_hbm.at[wid], idx1_v)
    pltpu.sync_copy(yt_hbm.at[wid], idx2_v)
    descs = []
    for j in range(BAG):
        descs.append(pltpu.async_copy(s1_hbm.at[idx1_v.at[j]], vals1_v.at[j], sem))
        descs.append(pltpu.async_copy(s2_hbm.at[idx2_v.at[j]], vals2_v.at[j], sem))
    for d in descs:
        d.wait()

    for k in range(ROWS_PER_WORKER // 16):
        sl = pl.ds(k * 16, 16)
        acc = vals1_v[0, sl] + vals2_v[0, sl]
        for j in range(1, BAG):
            acc = acc + vals1_v[j, sl]
            acc = acc + vals2_v[j, sl]
        out_v[sl] = acc

    pltpu.sync_copy(out_v, out_hbm.at[pl.ds(base, ROWS_PER_WORKER)])


@functools.lru_cache(maxsize=None)
def _sc_bag_sum():
    return functools.partial(
        pl.kernel,
        mesh=plsc.VectorSubcoreMesh(core_axis_name="c", subcore_axis_name="s"),
        out_type=jax.ShapeDtypeStruct((BATCH,), jnp.float32),
        scratch_types=[
            pltpu.VMEM((BAG, ROWS_PER_WORKER), jnp.int32),
            pltpu.VMEM((BAG, ROWS_PER_WORKER), jnp.int32),
            pltpu.VMEM((BAG, ROWS_PER_WORKER), jnp.float32),
            pltpu.VMEM((BAG, ROWS_PER_WORKER), jnp.float32),
            pltpu.VMEM((ROWS_PER_WORKER,), jnp.float32),
            pltpu.SemaphoreType.DMA,
        ],
    )(_sc_body)


def kernel(x, y, table_1, table_2, w1a, b1a, w1b, b1b, w2a, b2a, w2b, b2b):
    s1, s2 = _project_tables(table_1, table_2, w1a, w1b, w2a, w2b,
                             b1a, b1b, b2a, b2b)
    xt = x.reshape(NUM_WORKERS, ROWS_PER_WORKER, BAG).transpose(0, 2, 1)
    yt = y.reshape(NUM_WORKERS, ROWS_PER_WORKER, BAG).transpose(0, 2, 1)
    out = _sc_bag_sum()(s1, s2, xt, yt)
    return out.reshape(BATCH, 1)
